# baseline (speedup 1.0000x reference)
import functools

import jax
import jax.numpy as jnp
from jax import lax
from jax.experimental import pallas as pl
from jax.experimental.pallas import tpu as pltpu
from jax.experimental.pallas import tpu_sc as plsc

_SYM = (0, 2, 5, 8)
_BS = 8
_NP = 2048
_L = 16
_NC = 2
_NS = 16
_NW = _NC * _NS
_QSC = 128
_QTC = _NP - _QSC
_QPW = _QSC // _NW
_NRV = _NP // _L
_QG = _QSC // _NW
_SS = 8
_F32_BIG = 3.0e38


def _bcast_lane(vec, k):
    idx = jnp.full((_L, 1), k, dtype=jnp.int32)
    return lax.gather(
        vec, idx,
        lax.GatherDimensionNumbers(
            offset_dims=(), collapsed_slice_dims=(0,), start_index_map=(0,)),
        (1,), mode=lax.GatherScatterMode.PROMISE_IN_BOUNDS)


def _rot(vec, off):
    idx = ((lax.iota(jnp.int32, _L) + off) & 15)[:, None]
    return lax.gather(
        vec, idx,
        lax.GatherDimensionNumbers(
            offset_dims=(), collapsed_slice_dims=(0,), start_index_map=(0,)),
        (1,), mode=lax.GatherScatterMode.PROMISE_IN_BOUNDS)


def _sqrt16(x):
    xc = jnp.maximum(x, jnp.float32(1e-30))
    i = lax.bitcast_convert_type(xc, jnp.int32)
    y = lax.bitcast_convert_type(jnp.int32(0x5F3759DF) - (i >> 1), jnp.float32)
    half = jnp.float32(0.5) * xc
    for _ in range(3):
        y = y * (jnp.float32(1.5) - half * y * y)
    return jnp.maximum(x, jnp.float32(0.0)) * y


def _make_sc_kernel():
    mesh = plsc.VectorSubcoreMesh(core_axis_name="c", subcore_axis_name="s")

    @functools.partial(
        pl.kernel,
        mesh=mesh,
        compiler_params=pltpu.CompilerParams(needs_layout_passes=False),
        out_type=jax.ShapeDtypeStruct((_NW, _L), jnp.float32),
        scratch_types=[
            pltpu.VMEM((_BS * 3 * _NP,), jnp.float32),
            pltpu.VMEM((_NP,), jnp.float32),
            pltpu.VMEM((_BS * 3 * _SS,), jnp.float32),
            pltpu.VMEM((_BS * _L,), jnp.float32),
            pltpu.VMEM((_L,), jnp.float32),
        ],
    )
    def sck(tt_hbm, mq_hbm, h_hbm, out_hbm, ref_v, rsq_v, mp_v, h_v,
            res_v):
        cid = lax.axis_index("c")
        sid = lax.axis_index("s")
        wid = sid * _NC + cid
        qbase = wid * _QPW
        lanes = lax.iota(jnp.int32, _L)
        goff = qbase % _SS
        qal = qbase - goff

        pltpu.sync_copy(tt_hbm, ref_v)
        pltpu.sync_copy(mq_hbm.at[wid], mp_v)
        pltpu.sync_copy(h_hbm, h_v)

        def batch_body(b, res):
            hv = h_v[pl.ds(b * _L, _L)]
            tb = b * (3 * _NP)
            mb = b * (3 * _SS)
            r00 = _bcast_lane(hv, 0)
            r01 = _bcast_lane(hv, 1)
            r02 = _bcast_lane(hv, 2)
            tx = _bcast_lane(hv, 3)
            r10 = _bcast_lane(hv, 4)
            r11 = _bcast_lane(hv, 5)
            r12 = _bcast_lane(hv, 6)
            ty = _bcast_lane(hv, 7)
            r20 = _bcast_lane(hv, 8)
            r21 = _bcast_lane(hv, 9)
            r22 = _bcast_lane(hv, 10)
            tz = _bcast_lane(hv, 11)

            def rsq_body(j, carry):
                for u in range(4):
                    o = j * (4 * _L) + u * _L
                    rx = ref_v[pl.ds(tb + 0 * _NP + o, _L)]
                    ry = ref_v[pl.ds(tb + 1 * _NP + o, _L)]
                    rz = ref_v[pl.ds(tb + 2 * _NP + o, _L)]
                    rsq_v[pl.ds(o, _L)] = rx * rx + ry * ry + rz * rz
                return carry

            lax.fori_loop(0, _NRV // 4, rsq_body, 0)

            mx = mp_v[pl.ds(mb, _L)]
            my = mp_v[pl.ds(mb + _SS, _L)]
            mz = mp_v[pl.ds(mb + 2 * _SS, _L)]
            valid = lanes < jnp.int32(_QPW)
            tfx = r00 * mx + r01 * my + r02 * mz + tx
            tfy = r10 * mx + r11 * my + r12 * mz + ty
            tfz = r20 * mx + r21 * my + r22 * mz + tz
            n2x = jnp.float32(-2.0) * tfx
            n2y = jnp.float32(-2.0) * tfy
            n2z = jnp.float32(-2.0) * tfz
            qsq = tfx * tfx + tfy * tfy + tfz * tfz
            gx = _rot(ref_v[pl.ds(tb + 0 * _NP + qal, _L)], goff)
            gy = _rot(ref_v[pl.ds(tb + 1 * _NP + qal, _L)], goff)
            gz = _rot(ref_v[pl.ds(tb + 2 * _NP + qal, _L)], goff)
            dx = tfx - gx
            dy = tfy - gy
            dz = tfz - gz
            diag = jnp.where(valid, _sqrt16(dx * dx + dy * dy + dz * dz),
                             jnp.float32(0.0))

            minvec = jnp.full((_L,), _F32_BIG, jnp.float32)
            bxs = [_bcast_lane(n2x, q) for q in range(_QG)]
            bys = [_bcast_lane(n2y, q) for q in range(_QG)]
            bzs = [_bcast_lane(n2z, q) for q in range(_QG)]

            def nn_body(j, accs):
                o2 = j * (2 * _L)
                new = list(accs)
                for u in range(2):
                    oo = o2 + u * _L
                    rx = ref_v[pl.ds(tb + 0 * _NP + oo, _L)]
                    ry = ref_v[pl.ds(tb + 1 * _NP + oo, _L)]
                    rz = ref_v[pl.ds(tb + 2 * _NP + oo, _L)]
                    rq = rsq_v[pl.ds(oo, _L)]
                    for q in range(_QG):
                        d2 = rx * bxs[q] + ry * bys[q] + rz * bzs[q] + rq
                        new[q] = jnp.minimum(new[q], d2)
                return tuple(new)

            accs = lax.fori_loop(
                0, _NRV // 2, nn_body,
                tuple(jnp.full((_L,), _F32_BIG, jnp.float32)
                      for _ in range(_QG)))
            for q in range(_QG):
                m = jnp.min(accs[q])
                minvec = jnp.where(lanes == q, m, minvec)
            msum = jnp.where(valid, _sqrt16(minvec + qsq), jnp.float32(0.0))

            res = jnp.where(lanes == b, jnp.sum(msum), res)
            res = jnp.where(lanes == (b + _BS), jnp.sum(diag), res)
            return res

        res = lax.fori_loop(0, _BS, batch_body, jnp.zeros((_L,), jnp.float32))
        res_v[:] = res
        pltpu.sync_copy(res_v, out_hbm.at[wid])

    return sck


def _tc_body(tt_ref, mq_ref, tq_ref, m4_ref, out_ref):
    tt = tt_ref[0]
    mp3 = mq_ref[0]
    tq3 = tq_ref[0]
    m38 = m4_ref[0, :3, :]
    tvec = m4_ref[0, 3:4, :]
    tfa = jnp.dot(mp3, m38, preferred_element_type=jnp.float32) + tvec
    rsq = (tt[0:1, :] * tt[0:1, :] + tt[1:2, :] * tt[1:2, :]
           + tt[2:3, :] * tt[2:3, :])
    bt = jnp.concatenate(
        [jnp.float32(-2.0) * tt, rsq,
         jnp.zeros((4, _NP), jnp.float32)], axis=0)
    minv = jnp.full((_QTC,), _F32_BIG, jnp.float32)
    for rb in range(_NP // 512):
        btb = bt[:, rb * 512:(rb + 1) * 512]
        gb = lax.dot_general(tfa, btb, (((1,), (0,)), ((), ())),
                             preferred_element_type=jnp.float32)
        minv = jnp.minimum(minv, jnp.min(gb, axis=1))
    tf3 = tfa[:, :3]
    qsq = jnp.sum(tf3 * tf3, axis=1)
    dmin = jnp.sqrt(jnp.maximum(minv + qsq, jnp.float32(0.0)))
    dif = tf3 - tq3
    ddiag = jnp.sqrt(jnp.sum(dif * dif, axis=1))
    oii = lax.broadcasted_iota(jnp.int32, (1, 8, 128), 2)
    out_ref[...] = jnp.where(
        oii == 0, jnp.sum(dmin),
        jnp.where(oii == 1, jnp.sum(ddiag), jnp.float32(0.0)))


def _make_tc_kernel():
    return pl.pallas_call(
        _tc_body,
        grid=(_BS,),
        in_specs=[
            pl.BlockSpec((1, 3, _NP), lambda b: (b, 0, 0)),
            pl.BlockSpec((1, _QTC, 3), lambda b: (b, 0, 0)),
            pl.BlockSpec((1, _QTC, 3), lambda b: (b, 0, 0)),
            pl.BlockSpec((1, 4, 8), lambda b: (b, 0, 0)),
        ],
        out_specs=pl.BlockSpec((1, 8, 128), lambda b: (b, 0, 0)),
        out_shape=jax.ShapeDtypeStruct((_BS, 8, 128), jnp.float32),
    )


_SC_KERNEL = _make_sc_kernel()
_TC_KERNEL = _make_tc_kernel()


def kernel(target, model_points, idx, H):
    tt3 = jnp.transpose(target, (0, 2, 1))
    tt = tt3.reshape(_BS * 3 * _NP)
    msc = jnp.transpose(model_points[:, :_QSC, :], (0, 2, 1))
    msc = jnp.transpose(msc.reshape(_BS, 3, _NW, _QPW), (2, 0, 1, 3))
    msc = jnp.pad(msc, ((0, 0), (0, 0), (0, 0), (0, _SS - _QPW)))
    msc = msc.reshape(_NW, _BS * 3 * _SS)
    hf = H.reshape(_BS * _L)
    mq3 = model_points[:, _QSC:, :]
    tq3 = target[:, _QSC:, :]
    m48 = (jnp.zeros((_BS, 4, 8), jnp.float32)
           .at[:, :3, :3].set(jnp.transpose(H[:, :3, :3], (0, 2, 1)))
           .at[:, 3, :3].set(H[:, :3, 3])
           .at[:, 3, 3].set(1.0))
    tcout = _TC_KERNEL(tt3, mq3, tq3, m48)
    parts = _SC_KERNEL(tt, msc, hf)
    sums = jnp.sum(parts, axis=0)
    dmin = (sums[:_BS] + tcout[:, 0, 0]) / jnp.float32(_NP)
    ddiag = (sums[_BS:] + tcout[:, 0, 1]) / jnp.float32(_NP)
    sym = jnp.asarray(_SYM, dtype=idx.dtype)
    is_sym = jnp.any(idx[:, 0, None] == sym[None, :], axis=1)
    return jnp.where(is_sym, dmin, ddiag)


# speedup vs baseline: 1.0255x; 1.0255x over previous
import functools

import jax
import jax.numpy as jnp
from jax import lax
from jax.experimental import pallas as pl
from jax.experimental.pallas import tpu as pltpu
from jax.experimental.pallas import tpu_sc as plsc

_SYM = (0, 2, 5, 8)
_BS = 8
_NP = 2048
_L = 16
_NC = 2
_NS = 16
_NW = _NC * _NS
_QSC = 256
_QTC = _NP - _QSC
_QPW = _QSC // _NW
_NRV = _NP // _L
_QG = _QSC // _NW
_SS = 8
_F32_BIG = 3.0e38


def _bcast_lane(vec, k):
    idx = jnp.full((_L, 1), k, dtype=jnp.int32)
    return lax.gather(
        vec, idx,
        lax.GatherDimensionNumbers(
            offset_dims=(), collapsed_slice_dims=(0,), start_index_map=(0,)),
        (1,), mode=lax.GatherScatterMode.PROMISE_IN_BOUNDS)


def _rot(vec, off):
    idx = ((lax.iota(jnp.int32, _L) + off) & 15)[:, None]
    return lax.gather(
        vec, idx,
        lax.GatherDimensionNumbers(
            offset_dims=(), collapsed_slice_dims=(0,), start_index_map=(0,)),
        (1,), mode=lax.GatherScatterMode.PROMISE_IN_BOUNDS)


def _sqrt16(x):
    xc = jnp.maximum(x, jnp.float32(1e-30))
    i = lax.bitcast_convert_type(xc, jnp.int32)
    y = lax.bitcast_convert_type(jnp.int32(0x5F3759DF) - (i >> 1), jnp.float32)
    half = jnp.float32(0.5) * xc
    for _ in range(3):
        y = y * (jnp.float32(1.5) - half * y * y)
    return jnp.maximum(x, jnp.float32(0.0)) * y


def _make_sc_kernel():
    mesh = plsc.VectorSubcoreMesh(core_axis_name="c", subcore_axis_name="s")

    @functools.partial(
        pl.kernel,
        mesh=mesh,
        compiler_params=pltpu.CompilerParams(needs_layout_passes=False),
        out_type=jax.ShapeDtypeStruct((_NW, _L), jnp.float32),
        scratch_types=[
            pltpu.VMEM((_BS * 3 * _NP,), jnp.float32),
            pltpu.VMEM((_NP,), jnp.float32),
            pltpu.VMEM((_BS * 3 * _SS,), jnp.float32),
            pltpu.VMEM((_BS * _L,), jnp.float32),
            pltpu.VMEM((_L,), jnp.float32),
        ],
    )
    def sck(tt_hbm, mq_hbm, h_hbm, out_hbm, ref_v, rsq_v, mp_v, h_v,
            res_v):
        cid = lax.axis_index("c")
        sid = lax.axis_index("s")
        wid = sid * _NC + cid
        qbase = wid * _QPW
        lanes = lax.iota(jnp.int32, _L)
        goff = qbase % _SS
        qal = qbase - goff

        pltpu.sync_copy(tt_hbm, ref_v)
        pltpu.sync_copy(mq_hbm.at[wid], mp_v)
        pltpu.sync_copy(h_hbm, h_v)

        def batch_body(b, res):
            hv = h_v[pl.ds(b * _L, _L)]
            tb = b * (3 * _NP)
            mb = b * (3 * _SS)
            r00 = _bcast_lane(hv, 0)
            r01 = _bcast_lane(hv, 1)
            r02 = _bcast_lane(hv, 2)
            tx = _bcast_lane(hv, 3)
            r10 = _bcast_lane(hv, 4)
            r11 = _bcast_lane(hv, 5)
            r12 = _bcast_lane(hv, 6)
            ty = _bcast_lane(hv, 7)
            r20 = _bcast_lane(hv, 8)
            r21 = _bcast_lane(hv, 9)
            r22 = _bcast_lane(hv, 10)
            tz = _bcast_lane(hv, 11)

            def rsq_body(j, carry):
                for u in range(4):
                    o = j * (4 * _L) + u * _L
                    rx = ref_v[pl.ds(tb + 0 * _NP + o, _L)]
                    ry = ref_v[pl.ds(tb + 1 * _NP + o, _L)]
                    rz = ref_v[pl.ds(tb + 2 * _NP + o, _L)]
                    rsq_v[pl.ds(o, _L)] = rx * rx + ry * ry + rz * rz
                return carry

            lax.fori_loop(0, _NRV // 4, rsq_body, 0)

            mx = mp_v[pl.ds(mb, _L)]
            my = mp_v[pl.ds(mb + _SS, _L)]
            mz = mp_v[pl.ds(mb + 2 * _SS, _L)]
            valid = lanes < jnp.int32(_QPW)
            tfx = r00 * mx + r01 * my + r02 * mz + tx
            tfy = r10 * mx + r11 * my + r12 * mz + ty
            tfz = r20 * mx + r21 * my + r22 * mz + tz
            n2x = jnp.float32(-2.0) * tfx
            n2y = jnp.float32(-2.0) * tfy
            n2z = jnp.float32(-2.0) * tfz
            qsq = tfx * tfx + tfy * tfy + tfz * tfz
            gx = _rot(ref_v[pl.ds(tb + 0 * _NP + qal, _L)], goff)
            gy = _rot(ref_v[pl.ds(tb + 1 * _NP + qal, _L)], goff)
            gz = _rot(ref_v[pl.ds(tb + 2 * _NP + qal, _L)], goff)
            dx = tfx - gx
            dy = tfy - gy
            dz = tfz - gz
            diag = jnp.where(valid, _sqrt16(dx * dx + dy * dy + dz * dz),
                             jnp.float32(0.0))

            minvec = jnp.full((_L,), _F32_BIG, jnp.float32)
            bxs = [_bcast_lane(n2x, q) for q in range(_QG)]
            bys = [_bcast_lane(n2y, q) for q in range(_QG)]
            bzs = [_bcast_lane(n2z, q) for q in range(_QG)]

            def nn_body(j, accs):
                o2 = j * (2 * _L)
                new = list(accs)
                for u in range(2):
                    oo = o2 + u * _L
                    rx = ref_v[pl.ds(tb + 0 * _NP + oo, _L)]
                    ry = ref_v[pl.ds(tb + 1 * _NP + oo, _L)]
                    rz = ref_v[pl.ds(tb + 2 * _NP + oo, _L)]
                    rq = rsq_v[pl.ds(oo, _L)]
                    for q in range(_QG):
                        d2 = rx * bxs[q] + ry * bys[q] + rz * bzs[q] + rq
                        new[q] = jnp.minimum(new[q], d2)
                return tuple(new)

            accs = lax.fori_loop(
                0, _NRV // 2, nn_body,
                tuple(jnp.full((_L,), _F32_BIG, jnp.float32)
                      for _ in range(_QG)))
            for q in range(_QG):
                m = jnp.min(accs[q])
                minvec = jnp.where(lanes == q, m, minvec)
            msum = jnp.where(valid, _sqrt16(minvec + qsq), jnp.float32(0.0))

            res = jnp.where(lanes == b, jnp.sum(msum), res)
            res = jnp.where(lanes == (b + _BS), jnp.sum(diag), res)
            return res

        res = lax.fori_loop(0, _BS, batch_body, jnp.zeros((_L,), jnp.float32))
        res_v[:] = res
        pltpu.sync_copy(res_v, out_hbm.at[wid])

    return sck


def _tc_body(tt_ref, mq_ref, tq_ref, m4_ref, out_ref):
    tt = tt_ref[0]
    mp3 = mq_ref[0]
    tq3 = tq_ref[0]
    m38 = m4_ref[0, :3, :]
    tvec = m4_ref[0, 3:4, :]
    tfa = jnp.dot(mp3, m38, preferred_element_type=jnp.float32) + tvec
    rsq = (tt[0:1, :] * tt[0:1, :] + tt[1:2, :] * tt[1:2, :]
           + tt[2:3, :] * tt[2:3, :])
    bt = jnp.concatenate(
        [jnp.float32(-2.0) * tt, rsq,
         jnp.zeros((4, _NP), jnp.float32)], axis=0)
    minv = jnp.full((_QTC,), _F32_BIG, jnp.float32)
    for rb in range(_NP // 512):
        btb = bt[:, rb * 512:(rb + 1) * 512]
        gb = lax.dot_general(tfa, btb, (((1,), (0,)), ((), ())),
                             preferred_element_type=jnp.float32)
        minv = jnp.minimum(minv, jnp.min(gb, axis=1))
    tf3 = tfa[:, :3]
    qsq = jnp.sum(tf3 * tf3, axis=1)
    dmin = jnp.sqrt(jnp.maximum(minv + qsq, jnp.float32(0.0)))
    dif = tf3 - tq3
    ddiag = jnp.sqrt(jnp.sum(dif * dif, axis=1))
    oii = lax.broadcasted_iota(jnp.int32, (1, 8, 128), 2)
    out_ref[...] = jnp.where(
        oii == 0, jnp.sum(dmin),
        jnp.where(oii == 1, jnp.sum(ddiag), jnp.float32(0.0)))


def _make_tc_kernel():
    return pl.pallas_call(
        _tc_body,
        grid=(_BS,),
        in_specs=[
            pl.BlockSpec((1, 3, _NP), lambda b: (b, 0, 0)),
            pl.BlockSpec((1, _QTC, 3), lambda b: (b, 0, 0)),
            pl.BlockSpec((1, _QTC, 3), lambda b: (b, 0, 0)),
            pl.BlockSpec((1, 4, 8), lambda b: (b, 0, 0)),
        ],
        out_specs=pl.BlockSpec((1, 8, 128), lambda b: (b, 0, 0)),
        out_shape=jax.ShapeDtypeStruct((_BS, 8, 128), jnp.float32),
    )


_SC_KERNEL = _make_sc_kernel()
_TC_KERNEL = _make_tc_kernel()


def kernel(target, model_points, idx, H):
    tt3 = jnp.transpose(target, (0, 2, 1))
    tt = tt3.reshape(_BS * 3 * _NP)
    msc = jnp.transpose(model_points[:, :_QSC, :], (0, 2, 1))
    msc = jnp.transpose(msc.reshape(_BS, 3, _NW, _QPW), (2, 0, 1, 3))
    msc = jnp.pad(msc, ((0, 0), (0, 0), (0, 0), (0, _SS - _QPW)))
    msc = msc.reshape(_NW, _BS * 3 * _SS)
    hf = H.reshape(_BS * _L)
    mq3 = model_points[:, _QSC:, :]
    tq3 = target[:, _QSC:, :]
    m48 = (jnp.zeros((_BS, 4, 8), jnp.float32)
           .at[:, :3, :3].set(jnp.transpose(H[:, :3, :3], (0, 2, 1)))
           .at[:, 3, :3].set(H[:, :3, 3])
           .at[:, 3, 3].set(1.0))
    tcout = _TC_KERNEL(tt3, mq3, tq3, m48)
    parts = _SC_KERNEL(tt, msc, hf)
    sums = jnp.sum(parts, axis=0)
    dmin = (sums[:_BS] + tcout[:, 0, 0]) / jnp.float32(_NP)
    ddiag = (sums[_BS:] + tcout[:, 0, 1]) / jnp.float32(_NP)
    sym = jnp.asarray(_SYM, dtype=idx.dtype)
    is_sym = jnp.any(idx[:, 0, None] == sym[None, :], axis=1)
    return jnp.where(is_sym, dmin, ddiag)
